# baseline (speedup 1.0000x reference)
import functools

import jax
import jax.numpy as jnp
from jax import lax
from jax.experimental import pallas as pl
from jax.experimental.pallas import tpu as pltpu
from jax.experimental.pallas import tpu_sc as plsc

EMBED_DIM = 64
BATCH = 16384
HIST = 50
VOCAB = 1000000
TOTAL = BATCH * HIST

NUM_CORES = 2
NUM_SUBCORES = 16
NW = NUM_CORES * NUM_SUBCORES
COLS = BATCH // NW
NBUF = 2


def _build_gather():
    mesh = plsc.VectorSubcoreMesh(core_axis_name="c", subcore_axis_name="s")

    @functools.partial(
        pl.kernel,
        mesh=mesh,
        out_type=jax.ShapeDtypeStruct((TOTAL, EMBED_DIM), jnp.float32),
        scratch_types=[
            pltpu.VMEM((HIST, COLS), jnp.int32),
            pltpu.VMEM((COLS, EMBED_DIM), jnp.float32),
            pltpu.VMEM((COLS, EMBED_DIM), jnp.float32),
            pltpu.SemaphoreType.DMA,
            pltpu.SemaphoreType.DMA,
            pltpu.SemaphoreType.DMA,
            pltpu.SemaphoreType.DMA,
        ],
        compiler_params=pltpu.CompilerParams(use_tc_tiling_on_sc=False),
    )
    def gather_kernel(idx_hbm, table_hbm, out_hbm,
                      idx_all, rows0, rows1, sg0, sg1, so0, so1):
        wid = lax.axis_index("s") * NUM_CORES + lax.axis_index("c")
        col0 = wid * COLS

        rows = (rows0, rows1)
        sg = (sg0, sg1)
        so = (so0, so1)

        pltpu.sync_copy(idx_hbm.at[:, pl.ds(col0, COLS)], idx_all)

        pltpu.async_copy(table_hbm.at[idx_all.at[0]], rows0, sg0)
        pltpu.async_copy(table_hbm.at[idx_all.at[1]], rows1, sg1)

        def outer(i, carry):
            for b in range(NBUF):
                h = NBUF * i + b
                pltpu.make_async_copy(table_hbm.at[idx_all.at[h]],
                                      rows[b], sg[b]).wait()
                out_dma = pltpu.async_copy(
                    rows[b],
                    out_hbm.at[pl.ds(h * BATCH + col0, COLS)], so[b])
                out_dma.wait()

                @pl.when(h + NBUF < HIST)
                def _():
                    pltpu.async_copy(table_hbm.at[idx_all.at[h + NBUF]],
                                     rows[b], sg[b])
            return carry

        lax.fori_loop(0, HIST // NBUF, outer, 0)

    return gather_kernel


_gather = _build_gather()


@jax.jit
def kernel(data, table):
    idx_hm = data.T.astype(jnp.int32)
    flat = _gather(idx_hm, table)
    return flat.reshape(HIST, BATCH, EMBED_DIM).transpose(1, 0, 2)
